# baseline (speedup 1.0000x reference)
import functools

import jax
import jax.numpy as jnp
from jax import lax
from jax.experimental import pallas as pl
from jax.experimental.pallas import tpu as pltpu
from jax.experimental.pallas import tpu_sc as plsc

N = 10000
E = 160000
HID = 128
HEADS = 8
L = 4
RES_ALPHA = 0.1

EP = E + N
EPAD = 172032
SUB = 256
SUB_A = 64
EPT_A = EPAD // 32
NSUB_A = EPT_A // SUB_A
EPT_B = EPAD // 16
NSUB_B = EPT_B // SUB
RNG = 1280
SLAB = 1280
NRANGE = 4
DENR = 10240

_mesh = plsc.VectorSubcoreMesh(core_axis_name="c", subcore_axis_name="s")


def _permute(x, perm):
    dn = lax.GatherDimensionNumbers(offset_dims=(), collapsed_slice_dims=(0,),
                                    start_index_map=(0,))
    return lax.gather(x, perm[:, None], dn, (1,),
                      mode=lax.GatherScatterMode.PROMISE_IN_BOUNDS)


@functools.partial(
    pl.kernel,
    mesh=_mesh,
    out_type=[
        jax.ShapeDtypeStruct((EPAD, 128), jnp.float32),
        jax.ShapeDtypeStruct((2 * DENR, 128), jnp.float32),
    ],
    scratch_types=[
        pltpu.VMEM((SUB_A,), jnp.int32),
        pltpu.VMEM((SUB_A,), jnp.int32),
        pltpu.VMEM((SUB_A, 128), jnp.float32),
        pltpu.VMEM((SUB_A, 128), jnp.float32),
        pltpu.VMEM((SUB_A, 128), jnp.float32),
        pltpu.VMEM_SHARED((DENR, 128), jnp.float32),
        pltpu.SemaphoreType.DMA,
    ],
)
def _sc_a(src_hbm, dst_hbm, a128_hbm, zrow_hbm,
          ex_hbm, den_hbm,
          ssrc, sdst, srb, drb, exb128, dslab, sem):
    c = lax.axis_index("c")
    s = lax.axis_index("s")
    wid = s * 2 + c
    pltpu.sync_copy(zrow_hbm, dslab.at[pl.ds(s * 640, 640)])
    plsc.subcore_barrier()

    base = wid * EPT_A

    def sub_body(k, _):
        sb = base + k * SUB_A
        pltpu.sync_copy(src_hbm.at[pl.ds(sb, SUB_A)], ssrc)
        pltpu.sync_copy(dst_hbm.at[pl.ds(sb, SUB_A)], sdst)
        cpa = pltpu.async_copy(a128_hbm.at[ssrc], srb, sem)
        cpb = pltpu.async_copy(a128_hbm.at[sdst], drb, sem)
        cpa.wait()
        cpb.wait()

        def edge_body(j, _):
            x = srb[j, pl.ds(0, 16)] + drb[j, pl.ds(16, 16)]
            e = jnp.maximum(x, 0.2 * x) - drb[j, pl.ds(32, 16)]
            ex = jnp.exp(e)
            ex = jnp.where(sb + j < EP, ex, jnp.zeros((16,), jnp.float32))
            exb128[j, pl.ds(0, 16)] = ex
            return 0

        lax.fori_loop(0, SUB_A, edge_body, 0)
        pltpu.sync_copy(exb128, ex_hbm.at[pl.ds(sb, SUB_A)])
        pltpu.sync_copy(exb128, dslab.at[sdst], add=True)
        return 0

    lax.fori_loop(0, NSUB_A, sub_body, 0)
    plsc.subcore_barrier()
    pltpu.sync_copy(dslab.at[pl.ds(s * 640, 640)],
                    den_hbm.at[pl.ds(c * DENR + s * 640, 640)])


@functools.partial(
    pl.kernel,
    mesh=_mesh,
    out_type=jax.ShapeDtypeStruct((N * HEADS, HID), jnp.float32),
    scratch_types=[
        pltpu.VMEM((SUB,), jnp.int32),
        pltpu.VMEM((SUB,), jnp.int32),
        pltpu.VMEM((SUB + 16,), jnp.int32),
        pltpu.VMEM((SUB + 16,), jnp.int32),
        pltpu.VMEM((SUB + 16,), jnp.int32),
        pltpu.VMEM((32 * HEADS, HID), jnp.float32),
        pltpu.VMEM((32, 128), jnp.float32),
        pltpu.VMEM((32, 128), jnp.float32),
        pltpu.VMEM((32, 16), jnp.float32),
        pltpu.VMEM((32 * HEADS,), jnp.int32),
        pltpu.VMEM((32 * HEADS,), jnp.int32),
        pltpu.VMEM((32,), jnp.int32),
        pltpu.VMEM((32,), jnp.int32),
        pltpu.VMEM_SHARED((SLAB * HEADS, HID), jnp.float32),
        pltpu.SemaphoreType.DMA,
    ],
)
def _sc_b(src_hbm, dst_hbm, ex_hbm, dinv_hbm, hp_hbm, zrow_hbm,
          out_hbm,
          ssrc, sdst, csrc, cdst, ceid, rowb, exg, dvg, abuf,
          ihp, isc, i16e, i16d, slab, sem):
    c = lax.axis_index("c")
    s = lax.axis_index("s")
    iota = lax.iota(jnp.int32, 16)
    zi = jnp.zeros((16,), jnp.int32)

    for j in range(SUB // 16 + 1):
        csrc[pl.ds(j * 16, 16)] = zi
        cdst[pl.ds(j * 16, 16)] = zi
        ceid[pl.ds(j * 16, 16)] = zi
    for j in range(2 * HEADS):
        ihp[pl.ds(j * 16, 16)] = zi
        isc[pl.ds(j * 16, 16)] = zi
    for j in range(2):
        i16e[pl.ds(j * 16, 16)] = zi
        i16d[pl.ds(j * 16, 16)] = zi

    ebase = s * EPT_B

    def range_body(r, _):
        lo = (2 * r + c) * RNG
        hi = lo + RNG
        pltpu.sync_copy(zrow_hbm, slab.at[pl.ds(s * 640, 640)])
        plsc.subcore_barrier()

        def sub_body(k, _):
            sb = ebase + k * SUB
            pltpu.sync_copy(src_hbm.at[pl.ds(sb, SUB)], ssrc)
            pltpu.sync_copy(dst_hbm.at[pl.ds(sb, SUB)], sdst)
            cnt = jnp.int32(0)
            for j in range(SUB // 16):
                d16 = sdst[pl.ds(j * 16, 16)]
                s16 = ssrc[pl.ds(j * 16, 16)]
                m = (jnp.where(d16 >= lo, 1, 0)
                     * jnp.where(d16 < hi, 1, 0)) > 0
                mi = jnp.where(m, 1, 0)
                v = mi
                for step in (1, 2, 4, 8):
                    sh = _permute(v, jnp.maximum(iota - step, 0))
                    v = v + jnp.where(iota >= step, sh, 0)
                rank = v - mi
                inc = v[15]
                sel = zi
                for k2 in range(16):
                    mk = mi[k2]
                    rk = rank[k2]
                    cond = jnp.where(iota == rk, mk, 0) > 0
                    sel = jnp.where(cond, k2, sel)
                csrc[pl.ds(cnt, 16)] = _permute(s16, sel)
                cdst[pl.ds(cnt, 16)] = _permute(d16, sel)
                eidv = sb + j * 16 + iota
                ceid[pl.ds(cnt, 16)] = _permute(eidv, sel)
                cnt = cnt + inc

            def flush(g, _):
                gb = g * 32
                for u in range(2):
                    d16 = cdst[pl.ds(gb + u * 16, 16)]
                    s16 = csrc[pl.ds(gb + u * 16, 16)]
                    validv = (gb + u * 16 + iota) < cnt
                    ldv = jnp.where(validv, d16 - lo, zi)
                    for h in range(HEADS):
                        ihp[pl.ds(h * 32 + u * 16, 16)] = s16 * HEADS + h
                        isc[pl.ds(h * 32 + u * 16, 16)] = ldv * HEADS + h
                    i16e[pl.ds(u * 16, 16)] = ceid[pl.ds(gb + u * 16, 16)]
                    i16d[pl.ds(u * 16, 16)] = d16
                cp1 = pltpu.async_copy(hp_hbm.at[ihp], rowb, sem)
                cp2 = pltpu.async_copy(ex_hbm.at[i16e], exg, sem)
                cp3 = pltpu.async_copy(dinv_hbm.at[i16d], dvg, sem)
                cp1.wait()
                cp2.wait()
                cp3.wait()

                def alpha_body(e, _):
                    av = exg[e, pl.ds(0, 16)] * dvg[e, pl.ds(0, 16)]
                    av = jnp.where(gb + e < cnt, av,
                                   jnp.zeros((16,), jnp.float32))
                    abuf[e] = av
                    return 0

                lax.fori_loop(0, 32, alpha_body, 0)

                def scale(e2, _):
                    av = abuf[e2]
                    for h in range(HEADS):
                        a = av[h]
                        for kk in range(HID // 16):
                            rowb[h * 32 + e2, pl.ds(kk * 16, 16)] = (
                                rowb[h * 32 + e2, pl.ds(kk * 16, 16)] * a)
                    return 0

                lax.fori_loop(0, 32, scale, 0)
                pltpu.sync_copy(rowb, slab.at[isc], add=True)
                return 0

            ng = (cnt + 31) // 32
            lax.fori_loop(0, ng, flush, 0)
            return 0

        lax.fori_loop(0, NSUB_B, sub_body, 0)
        plsc.subcore_barrier()
        for q in range(5):
            row0 = s * 80 + q * 16

            @pl.when(lo + row0 + 16 <= N)
            def _():
                pltpu.sync_copy(slab.at[pl.ds(row0 * HEADS, 16 * HEADS)],
                                out_hbm.at[pl.ds((lo + row0) * HEADS,
                                                 16 * HEADS)])

        plsc.subcore_barrier()
        return 0

    lax.fori_loop(0, NRANGE, range_body, 0)


def _elu(z):
    return jnp.where(z > 0, z, jnp.exp(z) - 1.0)


def _project(h, wg_ref, asd_ref, hp_ref, a128_ref):
    for hh in range(HEADS):
        hp_ref[:, hh, :] = jnp.dot(h, wg_ref[:, hh * HID:(hh + 1) * HID],
                                   preferred_element_type=jnp.float32)
    esd = jnp.dot(h, asd_ref[...], preferred_element_type=jnp.float32)
    es = esd[:, 0:8]
    ed = esd[:, 8:16]
    x = es + ed
    t = jnp.maximum(x, 0.2 * x)
    z = jnp.zeros((h.shape[0], 80), jnp.float32)
    a128_ref[...] = jnp.concatenate([es, es, ed, ed, t, t, z], axis=1)


def _entry_body(x_ref, win_ref, bin_ref, wg_ref, asd_ref,
                h_ref, hp_ref, a128_ref):
    h = _elu(jnp.dot(x_ref[...], win_ref[...],
                     preferred_element_type=jnp.float32) + bin_ref[...])
    h_ref[...] = h
    _project(h, wg_ref, asd_ref, hp_ref, a128_ref)


def _ka_body(slab_ref, bg_ref, g_ref, sums_ref):
    acc = slab_ref[:, 0, :]
    for h in range(1, HEADS):
        acc = acc + slab_ref[:, h, :]
    g = acc + bg_ref[...]
    g_ref[...] = g
    s0 = jnp.sum(g, axis=0)
    s1 = jnp.sum(g * g, axis=0)
    blk = jnp.stack([s0, s1])

    @pl.when(pl.program_id(0) == 0)
    def _():
        sums_ref[...] = blk

    @pl.when(pl.program_id(0) != 0)
    def _():
        sums_ref[...] = sums_ref[...] + blk


def _bn_res(g_ref, sums_ref, hprev_ref, gam_ref, bet_ref):
    mu = sums_ref[0:1, :] / N
    var = sums_ref[1:2, :] / N - mu * mu
    rstd = lax.rsqrt(var + 1e-5)
    gn = (g_ref[...] - mu) * rstd * gam_ref[...] + bet_ref[...]
    return _elu((1.0 - RES_ALPHA) * gn + RES_ALPHA * hprev_ref[...])


def _kb_body(g_ref, sums_ref, hprev_ref, gam_ref, bet_ref, wg_ref, asd_ref,
             h_ref, hp_ref, a128_ref):
    h = _bn_res(g_ref, sums_ref, hprev_ref, gam_ref, bet_ref)
    h_ref[...] = h
    _project(h, wg_ref, asd_ref, hp_ref, a128_ref)


def _klast_body(g_ref, sums_ref, hprev_ref, gam_ref, bet_ref,
                w1_ref, b1_ref, w2_ref, b2_ref, o_ref):
    h = _bn_res(g_ref, sums_ref, hprev_ref, gam_ref, bet_ref)
    z = _elu(jnp.dot(h, w1_ref[...], preferred_element_type=jnp.float32)
             + b1_ref[...])
    o_ref[...] = jnp.dot(z, w2_ref[...],
                         preferred_element_type=jnp.float32) + b2_ref[...]


def _kd_body(d0_ref, d1_ref, dinv_ref):
    d = 0.125 / (d0_ref[:, 0:16] + d1_ref[:, 0:16] + 1e-16)
    z = jnp.zeros((d.shape[0], 112), jnp.float32)
    dinv_ref[...] = jnp.concatenate([d, z], axis=1)


_NB = 10
_BLK = N // _NB


def _row_spec(w):
    return pl.BlockSpec((_BLK, w), lambda i: (i, 0))


def _full_spec(shape):
    nd = len(shape)
    return pl.BlockSpec(shape, lambda i: (0,) * nd)


def _node_outs():
    return (
        [jax.ShapeDtypeStruct((N, HID), jnp.float32),
         jax.ShapeDtypeStruct((N, HEADS, HID), jnp.float32),
         jax.ShapeDtypeStruct((N, 128), jnp.float32)],
        [_row_spec(HID),
         pl.BlockSpec((_BLK, HEADS, HID), lambda i: (i, 0, 0)),
         _row_spec(128)],
    )


def kernel(x, edge_index, W_in, b_in, W_gat, att_src, att_dst, b_gat,
           bn_gamma, bn_beta, W1, b1, W2, b2):
    loops = jnp.arange(N, dtype=edge_index.dtype)
    src = jnp.concatenate([edge_index[0], loops,
                           jnp.zeros((EPAD - EP,), jnp.int32)])
    dst = jnp.concatenate([edge_index[1], loops,
                           jnp.zeros((EPAD - EP,), jnp.int32)])
    wg3 = W_gat.reshape(L, HID, HEADS, HID)
    a_s = jnp.einsum("ldhc,lhc->ldh", wg3, att_src)
    a_d = jnp.einsum("ldhc,lhc->ldh", wg3, att_dst)
    asd = jnp.concatenate([a_s, a_d], axis=2)
    zrow = jnp.zeros((640, HID), jnp.float32)
    w2p = jnp.zeros((W1.shape[1], 128), jnp.float32).at[:, :W2.shape[1]].set(W2)
    b2p = jnp.zeros((1, 128), jnp.float32).at[0, :W2.shape[1]].set(b2)

    outs, outspecs = _node_outs()
    h, hp, a128 = pl.pallas_call(
        _entry_body,
        grid=(_NB,),
        in_specs=[_row_spec(HID), _full_spec((HID, HID)),
                  _full_spec((HID,)), _full_spec((HID, HEADS * HID)),
                  _full_spec((HID, 16))],
        out_specs=outspecs,
        out_shape=outs,
    )(x, W_in, b_in, W_gat[0], asd[0])

    for i in range(L):
        ex, den = _sc_a(src, dst, a128, zrow)
        dinv = pl.pallas_call(
            _kd_body,
            grid=(_NB,),
            in_specs=[pl.BlockSpec((1024, 128), lambda i: (i, 0)),
                      pl.BlockSpec((1024, 128), lambda i: (i + _NB, 0))],
            out_specs=pl.BlockSpec((1024, 128), lambda i: (i, 0)),
            out_shape=jax.ShapeDtypeStruct((DENR, 128), jnp.float32),
        )(den, den)
        out_slab = _sc_b(src, dst, ex, dinv,
                         hp.reshape(N * HEADS, HID), zrow)
        g, sums = pl.pallas_call(
            _ka_body,
            grid=(_NB,),
            in_specs=[pl.BlockSpec((_BLK, HEADS, HID), lambda i: (i, 0, 0)),
                      _full_spec((1, HID))],
            out_specs=[_row_spec(HID),
                       pl.BlockSpec((2, HID), lambda i: (0, 0))],
            out_shape=[jax.ShapeDtypeStruct((N, HID), jnp.float32),
                       jax.ShapeDtypeStruct((2, HID), jnp.float32)],
        )(out_slab.reshape(N, HEADS, HID), b_gat[i][None, :])
        common = [_row_spec(HID), _full_spec((2, HID)), _row_spec(HID),
                  _full_spec((1, HID)), _full_spec((1, HID))]
        cargs = (g, sums, h, bn_gamma[i][None, :], bn_beta[i][None, :])
        if i < L - 1:
            h, hp, a128 = pl.pallas_call(
                _kb_body,
                grid=(_NB,),
                in_specs=common + [_full_spec((HID, HEADS * HID)),
                                   _full_spec((HID, 16))],
                out_specs=outspecs,
                out_shape=outs,
            )(*cargs, W_gat[i + 1], asd[i + 1])
        else:
            outp = pl.pallas_call(
                _klast_body,
                grid=(_NB,),
                in_specs=common + [_full_spec((HID, W1.shape[1])),
                                   _full_spec((W1.shape[1],)),
                                   _full_spec((W1.shape[1], 128)),
                                   _full_spec((1, 128))],
                out_specs=_row_spec(128),
                out_shape=jax.ShapeDtypeStruct((N, 128), jnp.float32),
            )(*cargs, W1, b1, w2p, b2p)
    return outp[:, :W2.shape[1]]


# speedup vs baseline: 1.0555x; 1.0555x over previous
import functools

import jax
import jax.numpy as jnp
from jax import lax
from jax.experimental import pallas as pl
from jax.experimental.pallas import tpu as pltpu
from jax.experimental.pallas import tpu_sc as plsc

N = 10000
E = 160000
HID = 128
HEADS = 8
L = 4
RES_ALPHA = 0.1

EP = E + N
EPAD = 172032
SUB = 256
SUB_A = 64
EPT_A = EPAD // 32
NSUB_A = EPT_A // SUB_A
EPT_B = EPAD // 16
NSUB_B = EPT_B // SUB
RNG = 1280
SLAB = 1280
NRANGE = 4
DENR = 10240

_mesh = plsc.VectorSubcoreMesh(core_axis_name="c", subcore_axis_name="s")


def _permute(x, perm):
    dn = lax.GatherDimensionNumbers(offset_dims=(), collapsed_slice_dims=(0,),
                                    start_index_map=(0,))
    return lax.gather(x, perm[:, None], dn, (1,),
                      mode=lax.GatherScatterMode.PROMISE_IN_BOUNDS)


@functools.partial(
    pl.kernel,
    mesh=_mesh,
    out_type=[
        jax.ShapeDtypeStruct((EPAD // 8, 128), jnp.float32),
        jax.ShapeDtypeStruct((2 * DENR, 128), jnp.float32),
    ],
    scratch_types=[
        pltpu.VMEM((SUB_A,), jnp.int32),
        pltpu.VMEM((SUB_A,), jnp.int32),
        pltpu.VMEM((SUB_A, 128), jnp.float32),
        pltpu.VMEM((SUB_A, 128), jnp.float32),
        pltpu.VMEM((SUB_A, 128), jnp.float32),
        pltpu.VMEM((SUB_A // 8, 128), jnp.float32),
        pltpu.VMEM_SHARED((DENR, 128), jnp.float32),
        pltpu.SemaphoreType.DMA,
    ],
)
def _sc_a(src_hbm, dst_hbm, a128_hbm, zrow_hbm,
          ex_hbm, den_hbm,
          ssrc, sdst, srb, drb, exb128, exbp, dslab, sem):
    c = lax.axis_index("c")
    s = lax.axis_index("s")
    wid = s * 2 + c
    pltpu.sync_copy(zrow_hbm, dslab.at[pl.ds(s * 640, 640)])
    plsc.subcore_barrier()

    base = wid * EPT_A

    def sub_body(k, _):
        sb = base + k * SUB_A
        pltpu.sync_copy(src_hbm.at[pl.ds(sb, SUB_A)], ssrc)
        pltpu.sync_copy(dst_hbm.at[pl.ds(sb, SUB_A)], sdst)
        cpa = pltpu.async_copy(a128_hbm.at[ssrc], srb, sem)
        cpb = pltpu.async_copy(a128_hbm.at[sdst], drb, sem)
        cpa.wait()
        cpb.wait()

        def edge_body(j, _):
            x = srb[j, pl.ds(0, 16)] + drb[j, pl.ds(16, 16)]
            e = jnp.maximum(x, 0.2 * x) - drb[j, pl.ds(32, 16)]
            ex = jnp.exp(e)
            ex = jnp.where(sb + j < EP, ex, jnp.zeros((16,), jnp.float32))
            exb128[j, pl.ds(0, 16)] = ex
            exbp[j // 8, pl.ds((j % 8) * 16, 16)] = ex
            return 0

        lax.fori_loop(0, SUB_A, edge_body, 0)
        pltpu.sync_copy(exbp, ex_hbm.at[pl.ds(pl.multiple_of(sb // 8, 8), SUB_A // 8)])
        pltpu.sync_copy(exb128, dslab.at[sdst], add=True)
        return 0

    lax.fori_loop(0, NSUB_A, sub_body, 0)
    plsc.subcore_barrier()
    pltpu.sync_copy(dslab.at[pl.ds(s * 640, 640)],
                    den_hbm.at[pl.ds(c * DENR + s * 640, 640)])


@functools.partial(
    pl.kernel,
    mesh=_mesh,
    out_type=jax.ShapeDtypeStruct((N * HEADS, HID), jnp.float32),
    scratch_types=[
        pltpu.VMEM((SUB,), jnp.int32),
        pltpu.VMEM((SUB,), jnp.int32),
        pltpu.VMEM((SUB + 16,), jnp.int32),
        pltpu.VMEM((SUB + 16,), jnp.int32),
        pltpu.VMEM((SUB + 16,), jnp.int32),
        pltpu.VMEM((16 * HEADS, HID), jnp.float32),
        pltpu.VMEM((16, 128), jnp.float32),
        pltpu.VMEM((16, 128), jnp.float32),
        pltpu.VMEM((16, 16), jnp.float32),
        pltpu.VMEM((16 * HEADS,), jnp.int32),
        pltpu.VMEM((16 * HEADS,), jnp.int32),
        pltpu.VMEM((16,), jnp.int32),
        pltpu.VMEM((16,), jnp.int32),
        pltpu.VMEM_SHARED((SLAB * HEADS, HID), jnp.float32),
        pltpu.SemaphoreType.DMA,
    ],
)
def _sc_b(src_hbm, dst_hbm, ex_hbm, dinv_hbm, hp_hbm, zrow_hbm,
          out_hbm,
          ssrc, sdst, csrc, cdst, ceid, rowb, exg, dvg, abuf,
          ihp, isc, i16e, i16d, slab, sem):
    c = lax.axis_index("c")
    s = lax.axis_index("s")
    iota = lax.iota(jnp.int32, 16)
    zi = jnp.zeros((16,), jnp.int32)

    for j in range(SUB // 16 + 1):
        csrc[pl.ds(j * 16, 16)] = zi
        cdst[pl.ds(j * 16, 16)] = zi
        ceid[pl.ds(j * 16, 16)] = zi
    for j in range(HEADS):
        ihp[pl.ds(j * 16, 16)] = zi
        isc[pl.ds(j * 16, 16)] = zi
    i16e[pl.ds(0, 16)] = zi
    i16d[pl.ds(0, 16)] = zi

    ebase = s * EPT_B

    def range_body(r, _):
        lo = (2 * r + c) * RNG
        hi = lo + RNG
        pltpu.sync_copy(zrow_hbm, slab.at[pl.ds(s * 640, 640)])
        plsc.subcore_barrier()

        def sub_body(k, _):
            sb = ebase + k * SUB
            pltpu.sync_copy(src_hbm.at[pl.ds(sb, SUB)], ssrc)
            pltpu.sync_copy(dst_hbm.at[pl.ds(sb, SUB)], sdst)
            cnt = jnp.int32(0)
            for j in range(SUB // 16):
                d16 = sdst[pl.ds(j * 16, 16)]
                s16 = ssrc[pl.ds(j * 16, 16)]
                m = (jnp.where(d16 >= lo, 1, 0)
                     * jnp.where(d16 < hi, 1, 0)) > 0
                mi = jnp.where(m, 1, 0)
                v = mi
                for step in (1, 2, 4, 8):
                    sh = _permute(v, jnp.maximum(iota - step, 0))
                    v = v + jnp.where(iota >= step, sh, 0)
                rank = v - mi
                inc = v[15]
                sel = zi
                for k2 in range(16):
                    mk = mi[k2]
                    rk = rank[k2]
                    cond = jnp.where(iota == rk, mk, 0) > 0
                    sel = jnp.where(cond, k2, sel)
                csrc[pl.ds(cnt, 16)] = _permute(s16, sel)
                cdst[pl.ds(cnt, 16)] = _permute(d16, sel)
                eidv = sb + j * 16 + iota
                ceid[pl.ds(cnt, 16)] = _permute(eidv, sel)
                cnt = cnt + inc

            def flush(g, _):
                gb = g * 16
                d16 = cdst[pl.ds(gb, 16)]
                s16 = csrc[pl.ds(gb, 16)]
                ceidv = ceid[pl.ds(gb, 16)]
                validv = (gb + iota) < cnt
                ldv = jnp.where(validv, d16 - lo, zi)
                for h in range(HEADS):
                    ihp[pl.ds(h * 16, 16)] = s16 * HEADS + h
                    isc[pl.ds(h * 16, 16)] = ldv * HEADS + h
                i16e[pl.ds(0, 16)] = lax.shift_right_logical(ceidv, 3)
                i16d[pl.ds(0, 16)] = d16
                cp1 = pltpu.async_copy(hp_hbm.at[ihp], rowb, sem)
                cp2 = pltpu.async_copy(ex_hbm.at[i16e], exg, sem)
                cp3 = pltpu.async_copy(dinv_hbm.at[i16d], dvg, sem)
                cp1.wait()
                cp2.wait()
                cp3.wait()

                for e in range(16):
                    ce = ceidv[e]
                    off = (ce & 7) * 16
                    av = exg[e, pl.ds(off, 16)] * dvg[e, pl.ds(0, 16)]
                    av = jnp.where(gb + e < cnt, av,
                                   jnp.zeros((16,), jnp.float32))
                    abuf[e] = av

                def scale(e2, _):
                    av = abuf[e2]
                    for h in range(HEADS):
                        a = av[h]
                        for kk in range(HID // 16):
                            rowb[h * 16 + e2, pl.ds(kk * 16, 16)] = (
                                rowb[h * 16 + e2, pl.ds(kk * 16, 16)] * a)
                    return 0

                lax.fori_loop(0, 16, scale, 0)
                pltpu.sync_copy(rowb, slab.at[isc], add=True)
                return 0

            ng = (cnt + 15) // 16
            lax.fori_loop(0, ng, flush, 0)
            return 0

        lax.fori_loop(0, NSUB_B, sub_body, 0)
        plsc.subcore_barrier()
        for q in range(5):
            row0 = s * 80 + q * 16

            @pl.when(lo + row0 + 16 <= N)
            def _():
                pltpu.sync_copy(slab.at[pl.ds(row0 * HEADS, 16 * HEADS)],
                                out_hbm.at[pl.ds((lo + row0) * HEADS,
                                                 16 * HEADS)])

        plsc.subcore_barrier()
        return 0

    lax.fori_loop(0, NRANGE, range_body, 0)


def _elu(z):
    return jnp.where(z > 0, z, jnp.exp(z) - 1.0)


def _project(h, wg_ref, asd_ref, hp_ref, a128_ref):
    for hh in range(HEADS):
        hp_ref[:, hh, :] = jnp.dot(h, wg_ref[:, hh * HID:(hh + 1) * HID],
                                   preferred_element_type=jnp.float32)
    esd = jnp.dot(h, asd_ref[...], preferred_element_type=jnp.float32)
    es = esd[:, 0:8]
    ed = esd[:, 8:16]
    x = es + ed
    t = jnp.maximum(x, 0.2 * x)
    z = jnp.zeros((h.shape[0], 80), jnp.float32)
    a128_ref[...] = jnp.concatenate([es, es, ed, ed, t, t, z], axis=1)


def _entry_body(x_ref, win_ref, bin_ref, wg_ref, asd_ref,
                h_ref, hp_ref, a128_ref):
    h = _elu(jnp.dot(x_ref[...], win_ref[...],
                     preferred_element_type=jnp.float32) + bin_ref[...])
    h_ref[...] = h
    _project(h, wg_ref, asd_ref, hp_ref, a128_ref)


def _ka_body(slab_ref, bg_ref, g_ref, sums_ref):
    acc = slab_ref[:, 0, :]
    for h in range(1, HEADS):
        acc = acc + slab_ref[:, h, :]
    g = acc + bg_ref[...]
    g_ref[...] = g
    s0 = jnp.sum(g, axis=0)
    s1 = jnp.sum(g * g, axis=0)
    blk = jnp.stack([s0, s1])

    @pl.when(pl.program_id(0) == 0)
    def _():
        sums_ref[...] = blk

    @pl.when(pl.program_id(0) != 0)
    def _():
        sums_ref[...] = sums_ref[...] + blk


def _bn_res(g_ref, sums_ref, hprev_ref, gam_ref, bet_ref):
    mu = sums_ref[0:1, :] / N
    var = sums_ref[1:2, :] / N - mu * mu
    rstd = lax.rsqrt(var + 1e-5)
    gn = (g_ref[...] - mu) * rstd * gam_ref[...] + bet_ref[...]
    return _elu((1.0 - RES_ALPHA) * gn + RES_ALPHA * hprev_ref[...])


def _kb_body(g_ref, sums_ref, hprev_ref, gam_ref, bet_ref, wg_ref, asd_ref,
             h_ref, hp_ref, a128_ref):
    h = _bn_res(g_ref, sums_ref, hprev_ref, gam_ref, bet_ref)
    h_ref[...] = h
    _project(h, wg_ref, asd_ref, hp_ref, a128_ref)


def _klast_body(g_ref, sums_ref, hprev_ref, gam_ref, bet_ref,
                w1_ref, b1_ref, w2_ref, b2_ref, o_ref):
    h = _bn_res(g_ref, sums_ref, hprev_ref, gam_ref, bet_ref)
    z = _elu(jnp.dot(h, w1_ref[...], preferred_element_type=jnp.float32)
             + b1_ref[...])
    o_ref[...] = jnp.dot(z, w2_ref[...],
                         preferred_element_type=jnp.float32) + b2_ref[...]


def _kd_body(d0_ref, d1_ref, dinv_ref):
    d = 0.125 / (d0_ref[:, 0:16] + d1_ref[:, 0:16] + 1e-16)
    z = jnp.zeros((d.shape[0], 112), jnp.float32)
    dinv_ref[...] = jnp.concatenate([d, z], axis=1)


_NB = 10
_BLK = N // _NB


def _row_spec(w):
    return pl.BlockSpec((_BLK, w), lambda i: (i, 0))


def _full_spec(shape):
    nd = len(shape)
    return pl.BlockSpec(shape, lambda i: (0,) * nd)


def _node_outs():
    return (
        [jax.ShapeDtypeStruct((N, HID), jnp.float32),
         jax.ShapeDtypeStruct((N, HEADS, HID), jnp.float32),
         jax.ShapeDtypeStruct((N, 128), jnp.float32)],
        [_row_spec(HID),
         pl.BlockSpec((_BLK, HEADS, HID), lambda i: (i, 0, 0)),
         _row_spec(128)],
    )


def kernel(x, edge_index, W_in, b_in, W_gat, att_src, att_dst, b_gat,
           bn_gamma, bn_beta, W1, b1, W2, b2):
    loops = jnp.arange(N, dtype=edge_index.dtype)
    src = jnp.concatenate([edge_index[0], loops,
                           jnp.zeros((EPAD - EP,), jnp.int32)])
    dst = jnp.concatenate([edge_index[1], loops,
                           jnp.zeros((EPAD - EP,), jnp.int32)])
    wg3 = W_gat.reshape(L, HID, HEADS, HID)
    a_s = jnp.einsum("ldhc,lhc->ldh", wg3, att_src)
    a_d = jnp.einsum("ldhc,lhc->ldh", wg3, att_dst)
    asd = jnp.concatenate([a_s, a_d], axis=2)
    zrow = jnp.zeros((640, HID), jnp.float32)
    w2p = jnp.zeros((W1.shape[1], 128), jnp.float32).at[:, :W2.shape[1]].set(W2)
    b2p = jnp.zeros((1, 128), jnp.float32).at[0, :W2.shape[1]].set(b2)

    outs, outspecs = _node_outs()
    h, hp, a128 = pl.pallas_call(
        _entry_body,
        grid=(_NB,),
        in_specs=[_row_spec(HID), _full_spec((HID, HID)),
                  _full_spec((HID,)), _full_spec((HID, HEADS * HID)),
                  _full_spec((HID, 16))],
        out_specs=outspecs,
        out_shape=outs,
    )(x, W_in, b_in, W_gat[0], asd[0])

    for i in range(L):
        ex, den = _sc_a(src, dst, a128, zrow)
        dinv = pl.pallas_call(
            _kd_body,
            grid=(_NB,),
            in_specs=[pl.BlockSpec((1024, 128), lambda i: (i, 0)),
                      pl.BlockSpec((1024, 128), lambda i: (i + _NB, 0))],
            out_specs=pl.BlockSpec((1024, 128), lambda i: (i, 0)),
            out_shape=jax.ShapeDtypeStruct((DENR, 128), jnp.float32),
        )(den, den)
        out_slab = _sc_b(src, dst, ex, dinv,
                         hp.reshape(N * HEADS, HID), zrow)
        g, sums = pl.pallas_call(
            _ka_body,
            grid=(_NB,),
            in_specs=[pl.BlockSpec((_BLK, HEADS, HID), lambda i: (i, 0, 0)),
                      _full_spec((1, HID))],
            out_specs=[_row_spec(HID),
                       pl.BlockSpec((2, HID), lambda i: (0, 0))],
            out_shape=[jax.ShapeDtypeStruct((N, HID), jnp.float32),
                       jax.ShapeDtypeStruct((2, HID), jnp.float32)],
        )(out_slab.reshape(N, HEADS, HID), b_gat[i][None, :])
        common = [_row_spec(HID), _full_spec((2, HID)), _row_spec(HID),
                  _full_spec((1, HID)), _full_spec((1, HID))]
        cargs = (g, sums, h, bn_gamma[i][None, :], bn_beta[i][None, :])
        if i < L - 1:
            h, hp, a128 = pl.pallas_call(
                _kb_body,
                grid=(_NB,),
                in_specs=common + [_full_spec((HID, HEADS * HID)),
                                   _full_spec((HID, 16))],
                out_specs=outspecs,
                out_shape=outs,
            )(*cargs, W_gat[i + 1], asd[i + 1])
        else:
            outp = pl.pallas_call(
                _klast_body,
                grid=(_NB,),
                in_specs=common + [_full_spec((HID, W1.shape[1])),
                                   _full_spec((W1.shape[1],)),
                                   _full_spec((W1.shape[1], 128)),
                                   _full_spec((1, 128))],
                out_specs=_row_spec(128),
                out_shape=jax.ShapeDtypeStruct((N, 128), jnp.float32),
            )(*cargs, W1, b1, w2p, b2p)
    return outp[:, :W2.shape[1]]


# speedup vs baseline: 1.0762x; 1.0196x over previous
import functools

import jax
import jax.numpy as jnp
from jax import lax
from jax.experimental import pallas as pl
from jax.experimental.pallas import tpu as pltpu
from jax.experimental.pallas import tpu_sc as plsc

N = 10000
E = 160000
HID = 128
HEADS = 8
L = 4
RES_ALPHA = 0.1

EP = E + N
EPAD = 172032
SUB = 256
SUB_A = 64
EPT_A = EPAD // 32
NSUB_A = EPT_A // SUB_A
EPT_B = EPAD // 16
NSUB_B = EPT_B // SUB
RNG = 1280
SLAB = 1280
NRANGE = 4
DENR = 10240

_mesh = plsc.VectorSubcoreMesh(core_axis_name="c", subcore_axis_name="s")


def _permute(x, perm):
    dn = lax.GatherDimensionNumbers(offset_dims=(), collapsed_slice_dims=(0,),
                                    start_index_map=(0,))
    return lax.gather(x, perm[:, None], dn, (1,),
                      mode=lax.GatherScatterMode.PROMISE_IN_BOUNDS)


@functools.partial(
    pl.kernel,
    mesh=_mesh,
    out_type=[
        jax.ShapeDtypeStruct((EPAD, 128), jnp.float32),
        jax.ShapeDtypeStruct((2 * DENR, 128), jnp.float32),
    ],
    scratch_types=[
        pltpu.VMEM((SUB_A,), jnp.int32),
        pltpu.VMEM((SUB_A,), jnp.int32),
        pltpu.VMEM((SUB_A, 128), jnp.float32),
        pltpu.VMEM((SUB_A, 128), jnp.float32),
        pltpu.VMEM((SUB_A, 128), jnp.float32),
        pltpu.VMEM_SHARED((DENR, 128), jnp.float32),
        pltpu.SemaphoreType.DMA,
    ],
)
def _sc_a(src_hbm, dst_hbm, a128_hbm, zrow_hbm,
          ex_hbm, den_hbm,
          ssrc, sdst, srb, drb, exb128, dslab, sem):
    c = lax.axis_index("c")
    s = lax.axis_index("s")
    wid = s * 2 + c
    pltpu.sync_copy(zrow_hbm, dslab.at[pl.ds(s * 640, 640)])
    plsc.subcore_barrier()

    base = wid * EPT_A

    def sub_body(k, _):
        sb = base + k * SUB_A
        pltpu.sync_copy(src_hbm.at[pl.ds(sb, SUB_A)], ssrc)
        pltpu.sync_copy(dst_hbm.at[pl.ds(sb, SUB_A)], sdst)
        cpa = pltpu.async_copy(a128_hbm.at[ssrc], srb, sem)
        cpb = pltpu.async_copy(a128_hbm.at[sdst], drb, sem)
        cpa.wait()
        cpb.wait()

        def edge_body(j, _):
            x = srb[j, pl.ds(0, 16)] + drb[j, pl.ds(16, 16)]
            e = jnp.maximum(x, 0.2 * x) - drb[j, pl.ds(32, 16)]
            ex = jnp.exp(e)
            ex = jnp.where(sb + j < EP, ex, jnp.zeros((16,), jnp.float32))
            exb128[j, pl.ds(0, 16)] = ex
            return 0

        lax.fori_loop(0, SUB_A, edge_body, 0)
        pltpu.sync_copy(exb128, ex_hbm.at[pl.ds(sb, SUB_A)])
        pltpu.sync_copy(exb128, dslab.at[sdst], add=True)
        return 0

    lax.fori_loop(0, NSUB_A, sub_body, 0)
    plsc.subcore_barrier()
    pltpu.sync_copy(dslab.at[pl.ds(s * 640, 640)],
                    den_hbm.at[pl.ds(c * DENR + s * 640, 640)])


@functools.partial(
    pl.kernel,
    mesh=_mesh,
    out_type=jax.ShapeDtypeStruct((N * HEADS, HID), jnp.float32),
    scratch_types=[
        pltpu.VMEM((SUB,), jnp.int32),
        pltpu.VMEM((SUB,), jnp.int32),
        pltpu.VMEM((SUB + 16,), jnp.int32),
        pltpu.VMEM((SUB + 16,), jnp.int32),
        pltpu.VMEM((SUB + 16,), jnp.int32),
        pltpu.VMEM((16 * HEADS, HID), jnp.float32),
        pltpu.VMEM((16, 128), jnp.float32),
        pltpu.VMEM((16, 128), jnp.float32),
        pltpu.VMEM((16, 16), jnp.float32),
        pltpu.VMEM((16 * HEADS,), jnp.int32),
        pltpu.VMEM((16 * HEADS,), jnp.int32),
        pltpu.VMEM((16,), jnp.int32),
        pltpu.VMEM((16,), jnp.int32),
        pltpu.VMEM_SHARED((SLAB * HEADS, HID), jnp.float32),
        pltpu.SemaphoreType.DMA,
    ],
)
def _sc_b(src_hbm, dst_hbm, ex_hbm, dinv_hbm, hp_hbm, zrow_hbm,
          out_hbm,
          ssrc, sdst, csrc, cdst, ceid, rowb, exg, dvg, abuf,
          ihp, isc, i16e, i16d, slab, sem):
    c = lax.axis_index("c")
    s = lax.axis_index("s")
    iota = lax.iota(jnp.int32, 16)
    zi = jnp.zeros((16,), jnp.int32)

    for j in range(SUB // 16 + 1):
        csrc[pl.ds(j * 16, 16)] = zi
        cdst[pl.ds(j * 16, 16)] = zi
        ceid[pl.ds(j * 16, 16)] = zi
    for j in range(HEADS):
        ihp[pl.ds(j * 16, 16)] = zi
        isc[pl.ds(j * 16, 16)] = zi
    i16e[pl.ds(0, 16)] = zi
    i16d[pl.ds(0, 16)] = zi

    ebase = s * EPT_B

    def range_body(r, _):
        lo = (2 * r + c) * RNG
        hi = lo + RNG
        pltpu.sync_copy(zrow_hbm, slab.at[pl.ds(s * 640, 640)])
        plsc.subcore_barrier()

        def sub_body(k, _):
            sb = ebase + k * SUB
            pltpu.sync_copy(src_hbm.at[pl.ds(sb, SUB)], ssrc)
            pltpu.sync_copy(dst_hbm.at[pl.ds(sb, SUB)], sdst)
            cnt = jnp.int32(0)
            for j in range(SUB // 16):
                d16 = sdst[pl.ds(j * 16, 16)]
                s16 = ssrc[pl.ds(j * 16, 16)]
                m = (jnp.where(d16 >= lo, 1, 0)
                     * jnp.where(d16 < hi, 1, 0)) > 0
                mi = jnp.where(m, 1, 0)
                v = mi
                for step in (1, 2, 4, 8):
                    sh = _permute(v, jnp.maximum(iota - step, 0))
                    v = v + jnp.where(iota >= step, sh, 0)
                rank = v - mi
                inc = v[15]
                sel = zi
                for k2 in range(16):
                    mk = mi[k2]
                    rk = rank[k2]
                    cond = jnp.where(iota == rk, mk, 0) > 0
                    sel = jnp.where(cond, k2, sel)
                csrc[pl.ds(cnt, 16)] = _permute(s16, sel)
                cdst[pl.ds(cnt, 16)] = _permute(d16, sel)
                eidv = sb + j * 16 + iota
                ceid[pl.ds(cnt, 16)] = _permute(eidv, sel)
                cnt = cnt + inc

            def flush(g, _):
                gb = g * 16
                d16 = cdst[pl.ds(gb, 16)]
                s16 = csrc[pl.ds(gb, 16)]
                ceidv = ceid[pl.ds(gb, 16)]
                validv = (gb + iota) < cnt
                ldv = jnp.where(validv, d16 - lo, zi)
                for h in range(HEADS):
                    ihp[pl.ds(h * 16, 16)] = s16 * HEADS + h
                    isc[pl.ds(h * 16, 16)] = ldv * HEADS + h
                i16e[pl.ds(0, 16)] = ceidv
                i16d[pl.ds(0, 16)] = d16
                cp1 = pltpu.async_copy(hp_hbm.at[ihp], rowb, sem)
                cp2 = pltpu.async_copy(ex_hbm.at[i16e], exg, sem)
                cp3 = pltpu.async_copy(dinv_hbm.at[i16d], dvg, sem)
                cp1.wait()
                cp2.wait()
                cp3.wait()

                def alpha_body(e, _):
                    av = exg[e, pl.ds(0, 16)] * dvg[e, pl.ds(0, 16)]
                    av = jnp.where(gb + e < cnt, av,
                                   jnp.zeros((16,), jnp.float32))
                    abuf[e] = av
                    return 0

                lax.fori_loop(0, 16, alpha_body, 0)

                def scale(e2, _):
                    av = abuf[e2]
                    for h in range(HEADS):
                        a = av[h]
                        for kk in range(HID // 16):
                            rowb[h * 16 + e2, pl.ds(kk * 16, 16)] = (
                                rowb[h * 16 + e2, pl.ds(kk * 16, 16)] * a)
                    return 0

                lax.fori_loop(0, 16, scale, 0)
                pltpu.sync_copy(rowb, slab.at[isc], add=True)
                return 0

            ng = (cnt + 15) // 16
            lax.fori_loop(0, ng, flush, 0)
            return 0

        lax.fori_loop(0, NSUB_B, sub_body, 0)
        plsc.subcore_barrier()
        for q in range(5):
            row0 = s * 80 + q * 16

            @pl.when(lo + row0 + 16 <= N)
            def _():
                pltpu.sync_copy(slab.at[pl.ds(row0 * HEADS, 16 * HEADS)],
                                out_hbm.at[pl.ds((lo + row0) * HEADS,
                                                 16 * HEADS)])

        plsc.subcore_barrier()
        return 0

    lax.fori_loop(0, NRANGE, range_body, 0)


def _elu(z):
    return jnp.where(z > 0, z, jnp.exp(z) - 1.0)


def _project(h, wg_ref, asd_ref, hp_ref, a128_ref):
    for hh in range(HEADS):
        hp_ref[:, hh, :] = jnp.dot(h, wg_ref[:, hh * HID:(hh + 1) * HID],
                                   preferred_element_type=jnp.float32)
    esd = jnp.dot(h, asd_ref[...], preferred_element_type=jnp.float32)
    es = esd[:, 0:8]
    ed = esd[:, 8:16]
    x = es + ed
    t = jnp.maximum(x, 0.2 * x)
    z = jnp.zeros((h.shape[0], 80), jnp.float32)
    a128_ref[...] = jnp.concatenate([es, es, ed, ed, t, t, z], axis=1)


def _entry_body(x_ref, win_ref, bin_ref, wg_ref, asd_ref,
                h_ref, hp_ref, a128_ref):
    h = _elu(jnp.dot(x_ref[...], win_ref[...],
                     preferred_element_type=jnp.float32) + bin_ref[...])
    h_ref[...] = h
    _project(h, wg_ref, asd_ref, hp_ref, a128_ref)


def _ka_body(slab_ref, bg_ref, g_ref, sums_ref):
    acc = slab_ref[:, 0, :]
    for h in range(1, HEADS):
        acc = acc + slab_ref[:, h, :]
    g = acc + bg_ref[...]
    g_ref[...] = g
    s0 = jnp.sum(g, axis=0)
    s1 = jnp.sum(g * g, axis=0)
    blk = jnp.stack([s0, s1])

    @pl.when(pl.program_id(0) == 0)
    def _():
        sums_ref[...] = blk

    @pl.when(pl.program_id(0) != 0)
    def _():
        sums_ref[...] = sums_ref[...] + blk


def _bn_res(g_ref, sums_ref, hprev_ref, gam_ref, bet_ref):
    mu = sums_ref[0:1, :] / N
    var = sums_ref[1:2, :] / N - mu * mu
    rstd = lax.rsqrt(var + 1e-5)
    gn = (g_ref[...] - mu) * rstd * gam_ref[...] + bet_ref[...]
    return _elu((1.0 - RES_ALPHA) * gn + RES_ALPHA * hprev_ref[...])


def _kb_body(g_ref, sums_ref, hprev_ref, gam_ref, bet_ref, wg_ref, asd_ref,
             h_ref, hp_ref, a128_ref):
    h = _bn_res(g_ref, sums_ref, hprev_ref, gam_ref, bet_ref)
    h_ref[...] = h
    _project(h, wg_ref, asd_ref, hp_ref, a128_ref)


def _klast_body(g_ref, sums_ref, hprev_ref, gam_ref, bet_ref,
                w1_ref, b1_ref, w2_ref, b2_ref, o_ref):
    h = _bn_res(g_ref, sums_ref, hprev_ref, gam_ref, bet_ref)
    z = _elu(jnp.dot(h, w1_ref[...], preferred_element_type=jnp.float32)
             + b1_ref[...])
    o_ref[...] = jnp.dot(z, w2_ref[...],
                         preferred_element_type=jnp.float32) + b2_ref[...]


def _kd_body(d0_ref, d1_ref, dinv_ref):
    d = 0.125 / (d0_ref[:, 0:16] + d1_ref[:, 0:16] + 1e-16)
    z = jnp.zeros((d.shape[0], 112), jnp.float32)
    dinv_ref[...] = jnp.concatenate([d, z], axis=1)


_NB = 10
_BLK = N // _NB


def _row_spec(w):
    return pl.BlockSpec((_BLK, w), lambda i: (i, 0))


def _full_spec(shape):
    nd = len(shape)
    return pl.BlockSpec(shape, lambda i: (0,) * nd)


def _node_outs():
    return (
        [jax.ShapeDtypeStruct((N, HID), jnp.float32),
         jax.ShapeDtypeStruct((N, HEADS, HID), jnp.float32),
         jax.ShapeDtypeStruct((N, 128), jnp.float32)],
        [_row_spec(HID),
         pl.BlockSpec((_BLK, HEADS, HID), lambda i: (i, 0, 0)),
         _row_spec(128)],
    )


def kernel(x, edge_index, W_in, b_in, W_gat, att_src, att_dst, b_gat,
           bn_gamma, bn_beta, W1, b1, W2, b2):
    loops = jnp.arange(N, dtype=edge_index.dtype)
    src = jnp.concatenate([edge_index[0], loops,
                           jnp.zeros((EPAD - EP,), jnp.int32)])
    dst = jnp.concatenate([edge_index[1], loops,
                           jnp.zeros((EPAD - EP,), jnp.int32)])
    wg3 = W_gat.reshape(L, HID, HEADS, HID)
    a_s = jnp.einsum("ldhc,lhc->ldh", wg3, att_src)
    a_d = jnp.einsum("ldhc,lhc->ldh", wg3, att_dst)
    asd = jnp.concatenate([a_s, a_d], axis=2)
    zrow = jnp.zeros((640, HID), jnp.float32)
    w2p = jnp.zeros((W1.shape[1], 128), jnp.float32).at[:, :W2.shape[1]].set(W2)
    b2p = jnp.zeros((1, 128), jnp.float32).at[0, :W2.shape[1]].set(b2)

    outs, outspecs = _node_outs()
    h, hp, a128 = pl.pallas_call(
        _entry_body,
        grid=(_NB,),
        in_specs=[_row_spec(HID), _full_spec((HID, HID)),
                  _full_spec((HID,)), _full_spec((HID, HEADS * HID)),
                  _full_spec((HID, 16))],
        out_specs=outspecs,
        out_shape=outs,
    )(x, W_in, b_in, W_gat[0], asd[0])

    for i in range(L):
        ex, den = _sc_a(src, dst, a128, zrow)
        dinv = pl.pallas_call(
            _kd_body,
            grid=(_NB,),
            in_specs=[pl.BlockSpec((1024, 128), lambda i: (i, 0)),
                      pl.BlockSpec((1024, 128), lambda i: (i + _NB, 0))],
            out_specs=pl.BlockSpec((1024, 128), lambda i: (i, 0)),
            out_shape=jax.ShapeDtypeStruct((DENR, 128), jnp.float32),
        )(den, den)
        out_slab = _sc_b(src, dst, ex, dinv,
                         hp.reshape(N * HEADS, HID), zrow)
        g, sums = pl.pallas_call(
            _ka_body,
            grid=(_NB,),
            in_specs=[pl.BlockSpec((_BLK, HEADS, HID), lambda i: (i, 0, 0)),
                      _full_spec((1, HID))],
            out_specs=[_row_spec(HID),
                       pl.BlockSpec((2, HID), lambda i: (0, 0))],
            out_shape=[jax.ShapeDtypeStruct((N, HID), jnp.float32),
                       jax.ShapeDtypeStruct((2, HID), jnp.float32)],
        )(out_slab.reshape(N, HEADS, HID), b_gat[i][None, :])
        common = [_row_spec(HID), _full_spec((2, HID)), _row_spec(HID),
                  _full_spec((1, HID)), _full_spec((1, HID))]
        cargs = (g, sums, h, bn_gamma[i][None, :], bn_beta[i][None, :])
        if i < L - 1:
            h, hp, a128 = pl.pallas_call(
                _kb_body,
                grid=(_NB,),
                in_specs=common + [_full_spec((HID, HEADS * HID)),
                                   _full_spec((HID, 16))],
                out_specs=outspecs,
                out_shape=outs,
            )(*cargs, W_gat[i + 1], asd[i + 1])
        else:
            outp = pl.pallas_call(
                _klast_body,
                grid=(_NB,),
                in_specs=common + [_full_spec((HID, W1.shape[1])),
                                   _full_spec((W1.shape[1],)),
                                   _full_spec((W1.shape[1], 128)),
                                   _full_spec((1, 128))],
                out_specs=_row_spec(128),
                out_shape=jax.ShapeDtypeStruct((N, 128), jnp.float32),
            )(*cargs, W1, b1, w2p, b2p)
    return outp[:, :W2.shape[1]]


# speedup vs baseline: 1.1088x; 1.0302x over previous
import functools

import jax
import jax.numpy as jnp
from jax import lax
from jax.experimental import pallas as pl
from jax.experimental.pallas import tpu as pltpu
from jax.experimental.pallas import tpu_sc as plsc

N = 10000
E = 160000
HID = 128
HEADS = 8
L = 4
RES_ALPHA = 0.1

EP = E + N
EPAD = 172032
SUB = 256
SUB_A = 64
EPT_A = EPAD // 32
NSUB_A = EPT_A // SUB_A
EPT_B = EPAD // 16
NSUB_B = EPT_B // SUB
RNG = 1280
SLAB = 1280
NRANGE = 4
DENR = 10240

_mesh = plsc.VectorSubcoreMesh(core_axis_name="c", subcore_axis_name="s")


def _permute(x, perm):
    dn = lax.GatherDimensionNumbers(offset_dims=(), collapsed_slice_dims=(0,),
                                    start_index_map=(0,))
    return lax.gather(x, perm[:, None], dn, (1,),
                      mode=lax.GatherScatterMode.PROMISE_IN_BOUNDS)


@functools.partial(
    pl.kernel,
    mesh=_mesh,
    out_type=[
        jax.ShapeDtypeStruct((EPAD, 128), jnp.float32),
        jax.ShapeDtypeStruct((2 * DENR, 128), jnp.float32),
    ],
    scratch_types=[
        pltpu.VMEM((SUB_A,), jnp.int32),
        pltpu.VMEM((SUB_A,), jnp.int32),
        pltpu.VMEM((SUB_A, 128), jnp.float32),
        pltpu.VMEM((SUB_A, 128), jnp.float32),
        pltpu.VMEM((SUB_A, 128), jnp.float32),
        pltpu.VMEM_SHARED((DENR, 128), jnp.float32),
        pltpu.SemaphoreType.DMA,
    ],
)
def _sc_a(src_hbm, dst_hbm, a128_hbm, zrow_hbm,
          ex_hbm, den_hbm,
          ssrc, sdst, srb, drb, exb128, dslab, sem):
    c = lax.axis_index("c")
    s = lax.axis_index("s")
    wid = s * 2 + c
    pltpu.sync_copy(zrow_hbm, dslab.at[pl.ds(s * 640, 640)])
    plsc.subcore_barrier()

    base = wid * EPT_A

    def sub_body(k, _):
        sb = base + k * SUB_A
        pltpu.sync_copy(src_hbm.at[pl.ds(sb, SUB_A)], ssrc)
        pltpu.sync_copy(dst_hbm.at[pl.ds(sb, SUB_A)], sdst)
        cpa = pltpu.async_copy(a128_hbm.at[ssrc], srb, sem)
        cpb = pltpu.async_copy(a128_hbm.at[sdst], drb, sem)
        cpa.wait()
        cpb.wait()

        def edge_body(j, _):
            x = srb[j, pl.ds(0, 16)] + drb[j, pl.ds(16, 16)]
            e = jnp.maximum(x, 0.2 * x) - drb[j, pl.ds(32, 16)]
            ex = jnp.exp(e)
            ex = jnp.where(sb + j < EP, ex, jnp.zeros((16,), jnp.float32))
            exb128[j, pl.ds(0, 16)] = ex
            return 0

        lax.fori_loop(0, SUB_A, edge_body, 0)
        pltpu.sync_copy(exb128, ex_hbm.at[pl.ds(sb, SUB_A)])
        pltpu.sync_copy(exb128, dslab.at[sdst], add=True)
        return 0

    lax.fori_loop(0, NSUB_A, sub_body, 0)
    plsc.subcore_barrier()
    pltpu.sync_copy(dslab.at[pl.ds(s * 640, 640)],
                    den_hbm.at[pl.ds(c * DENR + s * 640, 640)])


@functools.partial(
    pl.kernel,
    mesh=_mesh,
    out_type=jax.ShapeDtypeStruct((N * HEADS, HID), jnp.float32),
    scratch_types=[
        pltpu.VMEM((SUB,), jnp.int32),
        pltpu.VMEM((SUB,), jnp.int32),
        pltpu.VMEM((SUB + 16,), jnp.int32),
        pltpu.VMEM((SUB + 16,), jnp.int32),
        pltpu.VMEM((SUB + 16,), jnp.int32),
        pltpu.VMEM((16 * HEADS, HID), jnp.float32),
        pltpu.VMEM((16, 128), jnp.float32),
        pltpu.VMEM((16, 128), jnp.float32),
        pltpu.VMEM((16, 16), jnp.float32),
        pltpu.VMEM((16 * HEADS,), jnp.int32),
        pltpu.VMEM((16 * HEADS,), jnp.int32),
        pltpu.VMEM((16,), jnp.int32),
        pltpu.VMEM((16,), jnp.int32),
        pltpu.VMEM_SHARED((SLAB * HEADS, HID), jnp.float32),
        pltpu.SemaphoreType.DMA,
    ],
)
def _sc_b(src_hbm, dst_hbm, ex_hbm, dinv_hbm, hp_hbm, zrow_hbm,
          out_hbm,
          ssrc, sdst, csrc, cdst, ceid, rowb, exg, dvg, abuf,
          ihp, isc, i16e, i16d, slab, sem):
    c = lax.axis_index("c")
    s = lax.axis_index("s")
    iota = lax.iota(jnp.int32, 16)
    zi = jnp.zeros((16,), jnp.int32)

    for j in range(SUB // 16 + 1):
        csrc[pl.ds(j * 16, 16)] = zi
        cdst[pl.ds(j * 16, 16)] = zi
        ceid[pl.ds(j * 16, 16)] = zi
    for j in range(HEADS):
        ihp[pl.ds(j * 16, 16)] = zi
        isc[pl.ds(j * 16, 16)] = zi
    i16e[pl.ds(0, 16)] = zi
    i16d[pl.ds(0, 16)] = zi

    ebase = s * EPT_B

    def range_body(r, _):
        lo = (2 * r + c) * RNG
        hi = lo + RNG
        pltpu.sync_copy(zrow_hbm, slab.at[pl.ds(s * 640, 640)])
        plsc.subcore_barrier()

        def sub_body(k, _):
            sb = ebase + k * SUB
            cps = pltpu.async_copy(src_hbm.at[pl.ds(sb, SUB)], ssrc, sem)
            cpd = pltpu.async_copy(dst_hbm.at[pl.ds(sb, SUB)], sdst, sem)
            cps.wait()
            cpd.wait()
            cnt = jnp.int32(0)
            for j in range(SUB // 16):
                d16 = sdst[pl.ds(j * 16, 16)]
                s16 = ssrc[pl.ds(j * 16, 16)]
                m = (jnp.where(d16 >= lo, 1, 0)
                     * jnp.where(d16 < hi, 1, 0)) > 0
                mi = jnp.where(m, 1, 0)
                v = mi
                for step in (1, 2, 4, 8):
                    sh = _permute(v, jnp.maximum(iota - step, 0))
                    v = v + jnp.where(iota >= step, sh, 0)
                rank = v - mi
                inc = v[15]
                sel = zi
                for k2 in range(16):
                    mk = mi[k2]
                    rk = rank[k2]
                    cond = jnp.where(iota == rk, mk, 0) > 0
                    sel = jnp.where(cond, k2, sel)
                csrc[pl.ds(cnt, 16)] = _permute(s16, sel)
                cdst[pl.ds(cnt, 16)] = _permute(d16, sel)
                eidv = sb + j * 16 + iota
                ceid[pl.ds(cnt, 16)] = _permute(eidv, sel)
                cnt = cnt + inc

            def flush(g, _):
                gb = g * 16
                d16 = cdst[pl.ds(gb, 16)]
                s16 = csrc[pl.ds(gb, 16)]
                ceidv = ceid[pl.ds(gb, 16)]
                validv = (gb + iota) < cnt
                ldv = jnp.where(validv, d16 - lo, zi)
                for h in range(HEADS):
                    ihp[pl.ds(h * 16, 16)] = s16 * HEADS + h
                    isc[pl.ds(h * 16, 16)] = ldv * HEADS + h
                i16e[pl.ds(0, 16)] = ceidv
                i16d[pl.ds(0, 16)] = d16
                cp1 = pltpu.async_copy(hp_hbm.at[ihp], rowb, sem)
                cp2 = pltpu.async_copy(ex_hbm.at[i16e], exg, sem)
                cp3 = pltpu.async_copy(dinv_hbm.at[i16d], dvg, sem)
                cp1.wait()
                cp2.wait()
                cp3.wait()

                def alpha_body(e, _):
                    av = exg[e, pl.ds(0, 16)] * dvg[e, pl.ds(0, 16)]
                    av = jnp.where(gb + e < cnt, av,
                                   jnp.zeros((16,), jnp.float32))
                    abuf[e] = av
                    return 0

                lax.fori_loop(0, 16, alpha_body, 0)

                def scale(e2, _):
                    av = abuf[e2]
                    for h in range(HEADS):
                        a = av[h]
                        for kk in range(HID // 16):
                            rowb[h * 16 + e2, pl.ds(kk * 16, 16)] = (
                                rowb[h * 16 + e2, pl.ds(kk * 16, 16)] * a)
                    return 0

                lax.fori_loop(0, 16, scale, 0)
                pltpu.sync_copy(rowb, slab.at[isc], add=True)
                return 0

            ng = (cnt + 15) // 16
            lax.fori_loop(0, ng, flush, 0)
            return 0

        lax.fori_loop(0, NSUB_B, sub_body, 0)
        plsc.subcore_barrier()
        for q in range(5):
            row0 = s * 80 + q * 16

            @pl.when(lo + row0 + 16 <= N)
            def _():
                pltpu.sync_copy(slab.at[pl.ds(row0 * HEADS, 16 * HEADS)],
                                out_hbm.at[pl.ds((lo + row0) * HEADS,
                                                 16 * HEADS)])

        plsc.subcore_barrier()
        return 0

    lax.fori_loop(0, NRANGE, range_body, 0)


def _elu(z):
    return jnp.where(z > 0, z, jnp.exp(z) - 1.0)


def _project(h, wg_ref, asd_ref, hp_ref, a128_ref):
    for hh in range(HEADS):
        hp_ref[:, hh, :] = jnp.dot(h, wg_ref[:, hh * HID:(hh + 1) * HID],
                                   preferred_element_type=jnp.float32)
    esd = jnp.dot(h, asd_ref[...], preferred_element_type=jnp.float32)
    es = esd[:, 0:8]
    ed = esd[:, 8:16]
    x = es + ed
    t = jnp.maximum(x, 0.2 * x)
    z = jnp.zeros((h.shape[0], 80), jnp.float32)
    a128_ref[...] = jnp.concatenate([es, es, ed, ed, t, t, z], axis=1)


def _entry_body(x_ref, win_ref, bin_ref, wg_ref, asd_ref,
                h_ref, hp_ref, a128_ref):
    h = _elu(jnp.dot(x_ref[...], win_ref[...],
                     preferred_element_type=jnp.float32) + bin_ref[...])
    h_ref[...] = h
    _project(h, wg_ref, asd_ref, hp_ref, a128_ref)


def _ka_body(slab_ref, bg_ref, g_ref, sums_ref):
    acc = slab_ref[:, 0, :]
    for h in range(1, HEADS):
        acc = acc + slab_ref[:, h, :]
    g = acc + bg_ref[...]
    g_ref[...] = g
    s0 = jnp.sum(g, axis=0)
    s1 = jnp.sum(g * g, axis=0)
    blk = jnp.stack([s0, s1])

    @pl.when(pl.program_id(0) == 0)
    def _():
        sums_ref[...] = blk

    @pl.when(pl.program_id(0) != 0)
    def _():
        sums_ref[...] = sums_ref[...] + blk


def _bn_res(g_ref, sums_ref, hprev_ref, gam_ref, bet_ref):
    mu = sums_ref[0:1, :] / N
    var = sums_ref[1:2, :] / N - mu * mu
    rstd = lax.rsqrt(var + 1e-5)
    gn = (g_ref[...] - mu) * rstd * gam_ref[...] + bet_ref[...]
    return _elu((1.0 - RES_ALPHA) * gn + RES_ALPHA * hprev_ref[...])


def _kb_body(g_ref, sums_ref, hprev_ref, gam_ref, bet_ref, wg_ref, asd_ref,
             h_ref, hp_ref, a128_ref):
    h = _bn_res(g_ref, sums_ref, hprev_ref, gam_ref, bet_ref)
    h_ref[...] = h
    _project(h, wg_ref, asd_ref, hp_ref, a128_ref)


def _klast_body(g_ref, sums_ref, hprev_ref, gam_ref, bet_ref,
                w1_ref, b1_ref, w2_ref, b2_ref, o_ref):
    h = _bn_res(g_ref, sums_ref, hprev_ref, gam_ref, bet_ref)
    z = _elu(jnp.dot(h, w1_ref[...], preferred_element_type=jnp.float32)
             + b1_ref[...])
    o_ref[...] = jnp.dot(z, w2_ref[...],
                         preferred_element_type=jnp.float32) + b2_ref[...]


def _kd_body(d0_ref, d1_ref, dinv_ref):
    d = 0.125 / (d0_ref[:, 0:16] + d1_ref[:, 0:16] + 1e-16)
    z = jnp.zeros((d.shape[0], 112), jnp.float32)
    dinv_ref[...] = jnp.concatenate([d, z], axis=1)


_NB = 10
_BLK = N // _NB


def _row_spec(w):
    return pl.BlockSpec((_BLK, w), lambda i: (i, 0))


def _full_spec(shape):
    nd = len(shape)
    return pl.BlockSpec(shape, lambda i: (0,) * nd)


def _node_outs():
    return (
        [jax.ShapeDtypeStruct((N, HID), jnp.float32),
         jax.ShapeDtypeStruct((N, HEADS, HID), jnp.float32),
         jax.ShapeDtypeStruct((N, 128), jnp.float32)],
        [_row_spec(HID),
         pl.BlockSpec((_BLK, HEADS, HID), lambda i: (i, 0, 0)),
         _row_spec(128)],
    )


def kernel(x, edge_index, W_in, b_in, W_gat, att_src, att_dst, b_gat,
           bn_gamma, bn_beta, W1, b1, W2, b2):
    loops = jnp.arange(N, dtype=edge_index.dtype)
    src = jnp.concatenate([edge_index[0], loops,
                           jnp.zeros((EPAD - EP,), jnp.int32)])
    dst = jnp.concatenate([edge_index[1], loops,
                           jnp.zeros((EPAD - EP,), jnp.int32)])
    wg3 = W_gat.reshape(L, HID, HEADS, HID)
    a_s = jnp.einsum("ldhc,lhc->ldh", wg3, att_src)
    a_d = jnp.einsum("ldhc,lhc->ldh", wg3, att_dst)
    asd = jnp.concatenate([a_s, a_d], axis=2)
    zrow = jnp.zeros((640, HID), jnp.float32)
    w2p = jnp.zeros((W1.shape[1], 128), jnp.float32).at[:, :W2.shape[1]].set(W2)
    b2p = jnp.zeros((1, 128), jnp.float32).at[0, :W2.shape[1]].set(b2)

    outs, outspecs = _node_outs()
    h, hp, a128 = pl.pallas_call(
        _entry_body,
        grid=(_NB,),
        in_specs=[_row_spec(HID), _full_spec((HID, HID)),
                  _full_spec((HID,)), _full_spec((HID, HEADS * HID)),
                  _full_spec((HID, 16))],
        out_specs=outspecs,
        out_shape=outs,
    )(x, W_in, b_in, W_gat[0], asd[0])

    for i in range(L):
        ex, den = _sc_a(src, dst, a128, zrow)
        dinv = pl.pallas_call(
            _kd_body,
            grid=(_NB,),
            in_specs=[pl.BlockSpec((1024, 128), lambda i: (i, 0)),
                      pl.BlockSpec((1024, 128), lambda i: (i + _NB, 0))],
            out_specs=pl.BlockSpec((1024, 128), lambda i: (i, 0)),
            out_shape=jax.ShapeDtypeStruct((DENR, 128), jnp.float32),
        )(den, den)
        out_slab = _sc_b(src, dst, ex, dinv,
                         hp.reshape(N * HEADS, HID), zrow)
        g, sums = pl.pallas_call(
            _ka_body,
            grid=(_NB,),
            in_specs=[pl.BlockSpec((_BLK, HEADS, HID), lambda i: (i, 0, 0)),
                      _full_spec((1, HID))],
            out_specs=[_row_spec(HID),
                       pl.BlockSpec((2, HID), lambda i: (0, 0))],
            out_shape=[jax.ShapeDtypeStruct((N, HID), jnp.float32),
                       jax.ShapeDtypeStruct((2, HID), jnp.float32)],
        )(out_slab.reshape(N, HEADS, HID), b_gat[i][None, :])
        common = [_row_spec(HID), _full_spec((2, HID)), _row_spec(HID),
                  _full_spec((1, HID)), _full_spec((1, HID))]
        cargs = (g, sums, h, bn_gamma[i][None, :], bn_beta[i][None, :])
        if i < L - 1:
            h, hp, a128 = pl.pallas_call(
                _kb_body,
                grid=(_NB,),
                in_specs=common + [_full_spec((HID, HEADS * HID)),
                                   _full_spec((HID, 16))],
                out_specs=outspecs,
                out_shape=outs,
            )(*cargs, W_gat[i + 1], asd[i + 1])
        else:
            outp = pl.pallas_call(
                _klast_body,
                grid=(_NB,),
                in_specs=common + [_full_spec((HID, W1.shape[1])),
                                   _full_spec((W1.shape[1],)),
                                   _full_spec((W1.shape[1], 128)),
                                   _full_spec((1, 128))],
                out_specs=_row_spec(128),
                out_shape=jax.ShapeDtypeStruct((N, 128), jnp.float32),
            )(*cargs, W1, b1, w2p, b2p)
    return outp[:, :W2.shape[1]]


# speedup vs baseline: 1.1553x; 1.0419x over previous
import functools

import jax
import jax.numpy as jnp
from jax import lax
from jax.experimental import pallas as pl
from jax.experimental.pallas import tpu as pltpu
from jax.experimental.pallas import tpu_sc as plsc

N = 10000
E = 160000
HID = 128
HEADS = 8
L = 4
RES_ALPHA = 0.1

EP = E + N
EPAD = 172032
SUB = 256
SUB_A = 64
EPT_A = EPAD // 32
NSUB_A = EPT_A // SUB_A
EPT_B = EPAD // 16
NSUB_B = EPT_B // SUB
RNG = 1280
SLAB = 1280
NRANGE = 4
DENR = 10240

_mesh = plsc.VectorSubcoreMesh(core_axis_name="c", subcore_axis_name="s")


def _permute(x, perm):
    dn = lax.GatherDimensionNumbers(offset_dims=(), collapsed_slice_dims=(0,),
                                    start_index_map=(0,))
    return lax.gather(x, perm[:, None], dn, (1,),
                      mode=lax.GatherScatterMode.PROMISE_IN_BOUNDS)


@functools.partial(
    pl.kernel,
    mesh=_mesh,
    out_type=[
        jax.ShapeDtypeStruct((EPAD, 128), jnp.float32),
        jax.ShapeDtypeStruct((2 * DENR, 128), jnp.float32),
    ],
    scratch_types=[
        pltpu.VMEM((SUB_A,), jnp.int32),
        pltpu.VMEM((SUB_A,), jnp.int32),
        pltpu.VMEM((SUB_A, 128), jnp.float32),
        pltpu.VMEM((SUB_A, 128), jnp.float32),
        pltpu.VMEM((SUB_A, 128), jnp.float32),
        pltpu.VMEM_SHARED((DENR, 128), jnp.float32),
        pltpu.SemaphoreType.DMA,
    ],
)
def _sc_a(src_hbm, dst_hbm, a128_hbm, zrow_hbm,
          ex_hbm, den_hbm,
          ssrc, sdst, srb, drb, exb128, dslab, sem):
    c = lax.axis_index("c")
    s = lax.axis_index("s")
    wid = s * 2 + c
    pltpu.sync_copy(zrow_hbm, dslab.at[pl.ds(s * 640, 640)])
    plsc.subcore_barrier()

    base = wid * EPT_A

    def sub_body(k, _):
        sb = base + k * SUB_A
        pltpu.sync_copy(src_hbm.at[pl.ds(sb, SUB_A)], ssrc)
        pltpu.sync_copy(dst_hbm.at[pl.ds(sb, SUB_A)], sdst)
        cpa = pltpu.async_copy(a128_hbm.at[ssrc], srb, sem)
        cpb = pltpu.async_copy(a128_hbm.at[sdst], drb, sem)
        cpa.wait()
        cpb.wait()

        def edge_body(j, _):
            x = srb[j, pl.ds(0, 16)] + drb[j, pl.ds(16, 16)]
            e = jnp.maximum(x, 0.2 * x) - drb[j, pl.ds(32, 16)]
            ex = jnp.exp(e)
            ex = jnp.where(sb + j < EP, ex, jnp.zeros((16,), jnp.float32))
            exb128[j, pl.ds(0, 16)] = ex
            return 0

        lax.fori_loop(0, SUB_A, edge_body, 0)
        pltpu.sync_copy(exb128, ex_hbm.at[pl.ds(sb, SUB_A)])
        pltpu.sync_copy(exb128, dslab.at[sdst], add=True)
        return 0

    lax.fori_loop(0, NSUB_A, sub_body, 0)
    plsc.subcore_barrier()
    pltpu.sync_copy(dslab.at[pl.ds(s * 640, 640)],
                    den_hbm.at[pl.ds(c * DENR + s * 640, 640)])


@functools.partial(
    pl.kernel,
    mesh=_mesh,
    out_type=jax.ShapeDtypeStruct((N * HEADS, HID), jnp.float32),
    scratch_types=[
        pltpu.VMEM((SUB,), jnp.int32),
        pltpu.VMEM((SUB,), jnp.int32),
        pltpu.VMEM((SUB + 16,), jnp.int32),
        pltpu.VMEM((SUB + 16,), jnp.int32),
        pltpu.VMEM((SUB + 16,), jnp.int32),
        pltpu.VMEM((16 * HEADS, HID), jnp.float32),
        pltpu.VMEM((16, 128), jnp.float32),
        pltpu.VMEM((16, 128), jnp.float32),
        pltpu.VMEM((16, 16), jnp.float32),
        pltpu.VMEM((16 * HEADS,), jnp.int32),
        pltpu.VMEM((16 * HEADS,), jnp.int32),
        pltpu.VMEM((16,), jnp.int32),
        pltpu.VMEM((16,), jnp.int32),
        pltpu.VMEM_SHARED((SLAB * HEADS, HID), jnp.float32),
        pltpu.SemaphoreType.DMA,
    ],
)
def _sc_b(src_hbm, dst_hbm, ex_hbm, dinv_hbm, hp_hbm, zrow_hbm,
          out_hbm,
          ssrc, sdst, csrc, cdst, ceid, rowb, exg, dvg, abuf,
          ihp, isc, i16e, i16d, slab, sem):
    c = lax.axis_index("c")
    s = lax.axis_index("s")
    iota = lax.iota(jnp.int32, 16)
    zi = jnp.zeros((16,), jnp.int32)

    for j in range(SUB // 16 + 1):
        csrc[pl.ds(j * 16, 16)] = zi
        cdst[pl.ds(j * 16, 16)] = zi
        ceid[pl.ds(j * 16, 16)] = zi
    for j in range(HEADS):
        ihp[pl.ds(j * 16, 16)] = zi
        isc[pl.ds(j * 16, 16)] = zi
    i16e[pl.ds(0, 16)] = zi
    i16d[pl.ds(0, 16)] = zi

    ebase = s * EPT_B

    def range_body(r, _):
        lo = (2 * r + c) * RNG
        hi = lo + RNG
        pltpu.sync_copy(zrow_hbm, slab.at[pl.ds(s * 640, 640)])
        plsc.subcore_barrier()
        pltpu.async_copy(src_hbm.at[pl.ds(ebase, SUB)], ssrc, sem)
        pltpu.async_copy(dst_hbm.at[pl.ds(ebase, SUB)], sdst, sem)

        def sub_body(k, _):
            sb = ebase + k * SUB
            pltpu.make_async_copy(src_hbm.at[pl.ds(sb, SUB)], ssrc, sem).wait()
            pltpu.make_async_copy(dst_hbm.at[pl.ds(sb, SUB)], sdst, sem).wait()
            cnt = jnp.int32(0)
            for j in range(SUB // 16):
                d16 = sdst[pl.ds(j * 16, 16)]
                s16 = ssrc[pl.ds(j * 16, 16)]
                m = (jnp.where(d16 >= lo, 1, 0)
                     * jnp.where(d16 < hi, 1, 0)) > 0
                mi = jnp.where(m, 1, 0)
                v = mi
                for step in (1, 2, 4, 8):
                    sh = _permute(v, jnp.maximum(iota - step, 0))
                    v = v + jnp.where(iota >= step, sh, 0)
                rank = v - mi
                inc = v[15]
                sel = zi
                for k2 in range(16):
                    mk = mi[k2]
                    rk = rank[k2]
                    cond = jnp.where(iota == rk, mk, 0) > 0
                    sel = jnp.where(cond, k2, sel)
                csrc[pl.ds(cnt, 16)] = _permute(s16, sel)
                cdst[pl.ds(cnt, 16)] = _permute(d16, sel)
                eidv = sb + j * 16 + iota
                ceid[pl.ds(cnt, 16)] = _permute(eidv, sel)
                cnt = cnt + inc

            @pl.when(k + 1 < NSUB_B)
            def _():
                sbn = ebase + (k + 1) * SUB
                pltpu.async_copy(src_hbm.at[pl.ds(sbn, SUB)], ssrc, sem)
                pltpu.async_copy(dst_hbm.at[pl.ds(sbn, SUB)], sdst, sem)

            def flush(g, _):
                gb = g * 16
                d16 = cdst[pl.ds(gb, 16)]
                s16 = csrc[pl.ds(gb, 16)]
                ceidv = ceid[pl.ds(gb, 16)]
                validv = (gb + iota) < cnt
                ldv = jnp.where(validv, d16 - lo, zi)
                for h in range(HEADS):
                    ihp[pl.ds(h * 16, 16)] = s16 * HEADS + h
                    isc[pl.ds(h * 16, 16)] = ldv * HEADS + h
                i16e[pl.ds(0, 16)] = ceidv
                i16d[pl.ds(0, 16)] = d16
                cp1 = pltpu.async_copy(hp_hbm.at[ihp], rowb, sem)
                cp2 = pltpu.async_copy(ex_hbm.at[i16e], exg, sem)
                cp3 = pltpu.async_copy(dinv_hbm.at[i16d], dvg, sem)
                cp1.wait()
                cp2.wait()
                cp3.wait()

                def alpha_body(e, _):
                    av = exg[e, pl.ds(0, 16)] * dvg[e, pl.ds(0, 16)]
                    av = jnp.where(gb + e < cnt, av,
                                   jnp.zeros((16,), jnp.float32))
                    abuf[e] = av
                    return 0

                lax.fori_loop(0, 16, alpha_body, 0)

                def scale(e2, _):
                    av = abuf[e2]
                    for h in range(HEADS):
                        a = av[h]
                        for kk in range(HID // 16):
                            rowb[h * 16 + e2, pl.ds(kk * 16, 16)] = (
                                rowb[h * 16 + e2, pl.ds(kk * 16, 16)] * a)
                    return 0

                lax.fori_loop(0, 16, scale, 0)
                pltpu.sync_copy(rowb, slab.at[isc], add=True)
                return 0

            ng = (cnt + 15) // 16
            lax.fori_loop(0, ng, flush, 0)
            return 0

        lax.fori_loop(0, NSUB_B, sub_body, 0)
        plsc.subcore_barrier()
        for q in range(5):
            row0 = s * 80 + q * 16

            @pl.when(lo + row0 + 16 <= N)
            def _():
                pltpu.sync_copy(slab.at[pl.ds(row0 * HEADS, 16 * HEADS)],
                                out_hbm.at[pl.ds((lo + row0) * HEADS,
                                                 16 * HEADS)])

        plsc.subcore_barrier()
        return 0

    lax.fori_loop(0, NRANGE, range_body, 0)


def _elu(z):
    return jnp.where(z > 0, z, jnp.exp(z) - 1.0)


def _project(h, wg_ref, asd_ref, hp_ref, a128_ref):
    for hh in range(HEADS):
        hp_ref[:, hh, :] = jnp.dot(h, wg_ref[:, hh * HID:(hh + 1) * HID],
                                   preferred_element_type=jnp.float32)
    esd = jnp.dot(h, asd_ref[...], preferred_element_type=jnp.float32)
    es = esd[:, 0:8]
    ed = esd[:, 8:16]
    x = es + ed
    t = jnp.maximum(x, 0.2 * x)
    z = jnp.zeros((h.shape[0], 80), jnp.float32)
    a128_ref[...] = jnp.concatenate([es, es, ed, ed, t, t, z], axis=1)


def _entry_body(x_ref, win_ref, bin_ref, wg_ref, asd_ref,
                h_ref, hp_ref, a128_ref):
    h = _elu(jnp.dot(x_ref[...], win_ref[...],
                     preferred_element_type=jnp.float32) + bin_ref[...])
    h_ref[...] = h
    _project(h, wg_ref, asd_ref, hp_ref, a128_ref)


def _ka_body(slab_ref, bg_ref, g_ref, sums_ref):
    acc = slab_ref[:, 0, :]
    for h in range(1, HEADS):
        acc = acc + slab_ref[:, h, :]
    g = acc + bg_ref[...]
    g_ref[...] = g
    s0 = jnp.sum(g, axis=0)
    s1 = jnp.sum(g * g, axis=0)
    blk = jnp.stack([s0, s1])

    @pl.when(pl.program_id(0) == 0)
    def _():
        sums_ref[...] = blk

    @pl.when(pl.program_id(0) != 0)
    def _():
        sums_ref[...] = sums_ref[...] + blk


def _bn_res(g_ref, sums_ref, hprev_ref, gam_ref, bet_ref):
    mu = sums_ref[0:1, :] / N
    var = sums_ref[1:2, :] / N - mu * mu
    rstd = lax.rsqrt(var + 1e-5)
    gn = (g_ref[...] - mu) * rstd * gam_ref[...] + bet_ref[...]
    return _elu((1.0 - RES_ALPHA) * gn + RES_ALPHA * hprev_ref[...])


def _kb_body(g_ref, sums_ref, hprev_ref, gam_ref, bet_ref, wg_ref, asd_ref,
             h_ref, hp_ref, a128_ref):
    h = _bn_res(g_ref, sums_ref, hprev_ref, gam_ref, bet_ref)
    h_ref[...] = h
    _project(h, wg_ref, asd_ref, hp_ref, a128_ref)


def _klast_body(g_ref, sums_ref, hprev_ref, gam_ref, bet_ref,
                w1_ref, b1_ref, w2_ref, b2_ref, o_ref):
    h = _bn_res(g_ref, sums_ref, hprev_ref, gam_ref, bet_ref)
    z = _elu(jnp.dot(h, w1_ref[...], preferred_element_type=jnp.float32)
             + b1_ref[...])
    o_ref[...] = jnp.dot(z, w2_ref[...],
                         preferred_element_type=jnp.float32) + b2_ref[...]


def _kd_body(d0_ref, d1_ref, dinv_ref):
    d = 0.125 / (d0_ref[:, 0:16] + d1_ref[:, 0:16] + 1e-16)
    z = jnp.zeros((d.shape[0], 112), jnp.float32)
    dinv_ref[...] = jnp.concatenate([d, z], axis=1)


_NB = 10
_BLK = N // _NB


def _row_spec(w):
    return pl.BlockSpec((_BLK, w), lambda i: (i, 0))


def _full_spec(shape):
    nd = len(shape)
    return pl.BlockSpec(shape, lambda i: (0,) * nd)


def _node_outs():
    return (
        [jax.ShapeDtypeStruct((N, HID), jnp.float32),
         jax.ShapeDtypeStruct((N, HEADS, HID), jnp.float32),
         jax.ShapeDtypeStruct((N, 128), jnp.float32)],
        [_row_spec(HID),
         pl.BlockSpec((_BLK, HEADS, HID), lambda i: (i, 0, 0)),
         _row_spec(128)],
    )


def kernel(x, edge_index, W_in, b_in, W_gat, att_src, att_dst, b_gat,
           bn_gamma, bn_beta, W1, b1, W2, b2):
    loops = jnp.arange(N, dtype=edge_index.dtype)
    src = jnp.concatenate([edge_index[0], loops,
                           jnp.zeros((EPAD - EP,), jnp.int32)])
    dst = jnp.concatenate([edge_index[1], loops,
                           jnp.zeros((EPAD - EP,), jnp.int32)])
    wg3 = W_gat.reshape(L, HID, HEADS, HID)
    a_s = jnp.einsum("ldhc,lhc->ldh", wg3, att_src)
    a_d = jnp.einsum("ldhc,lhc->ldh", wg3, att_dst)
    asd = jnp.concatenate([a_s, a_d], axis=2)
    zrow = jnp.zeros((640, HID), jnp.float32)
    w2p = jnp.zeros((W1.shape[1], 128), jnp.float32).at[:, :W2.shape[1]].set(W2)
    b2p = jnp.zeros((1, 128), jnp.float32).at[0, :W2.shape[1]].set(b2)

    outs, outspecs = _node_outs()
    h, hp, a128 = pl.pallas_call(
        _entry_body,
        grid=(_NB,),
        in_specs=[_row_spec(HID), _full_spec((HID, HID)),
                  _full_spec((HID,)), _full_spec((HID, HEADS * HID)),
                  _full_spec((HID, 16))],
        out_specs=outspecs,
        out_shape=outs,
    )(x, W_in, b_in, W_gat[0], asd[0])

    for i in range(L):
        ex, den = _sc_a(src, dst, a128, zrow)
        dinv = pl.pallas_call(
            _kd_body,
            grid=(_NB,),
            in_specs=[pl.BlockSpec((1024, 128), lambda i: (i, 0)),
                      pl.BlockSpec((1024, 128), lambda i: (i + _NB, 0))],
            out_specs=pl.BlockSpec((1024, 128), lambda i: (i, 0)),
            out_shape=jax.ShapeDtypeStruct((DENR, 128), jnp.float32),
        )(den, den)
        out_slab = _sc_b(src, dst, ex, dinv,
                         hp.reshape(N * HEADS, HID), zrow)
        g, sums = pl.pallas_call(
            _ka_body,
            grid=(_NB,),
            in_specs=[pl.BlockSpec((_BLK, HEADS, HID), lambda i: (i, 0, 0)),
                      _full_spec((1, HID))],
            out_specs=[_row_spec(HID),
                       pl.BlockSpec((2, HID), lambda i: (0, 0))],
            out_shape=[jax.ShapeDtypeStruct((N, HID), jnp.float32),
                       jax.ShapeDtypeStruct((2, HID), jnp.float32)],
        )(out_slab.reshape(N, HEADS, HID), b_gat[i][None, :])
        common = [_row_spec(HID), _full_spec((2, HID)), _row_spec(HID),
                  _full_spec((1, HID)), _full_spec((1, HID))]
        cargs = (g, sums, h, bn_gamma[i][None, :], bn_beta[i][None, :])
        if i < L - 1:
            h, hp, a128 = pl.pallas_call(
                _kb_body,
                grid=(_NB,),
                in_specs=common + [_full_spec((HID, HEADS * HID)),
                                   _full_spec((HID, 16))],
                out_specs=outspecs,
                out_shape=outs,
            )(*cargs, W_gat[i + 1], asd[i + 1])
        else:
            outp = pl.pallas_call(
                _klast_body,
                grid=(_NB,),
                in_specs=common + [_full_spec((HID, W1.shape[1])),
                                   _full_spec((W1.shape[1],)),
                                   _full_spec((W1.shape[1], 128)),
                                   _full_spec((1, 128))],
                out_specs=_row_spec(128),
                out_shape=jax.ShapeDtypeStruct((N, 128), jnp.float32),
            )(*cargs, W1, b1, w2p, b2p)
    return outp[:, :W2.shape[1]]
